# baseline (speedup 1.0000x reference)
import jax
import jax.numpy as jnp
from jax import lax
from jax.experimental import pallas as pl
from jax.experimental.pallas import tpu as pltpu
from jax.experimental.pallas import tpu_sc as plsc

N = 10000
E = 160000
D = 256
HD = 128
KSTEPS = 2
P = 2.5

NTILES = 16
N_PAD = 10240
NPT = N_PAD // NTILES
E_T = 10240
E_PAD = E_T * NTILES
CH = 64
NCH = E_T // CH
NPAIR = NCH // 2
WB = 16
NWB = NPT // WB
NPADROWS = N_PAD - N


def _rsqrt_newton(x, iters):
    xi = lax.bitcast_convert_type(x, jnp.int32)
    yi = jnp.int32(0x5F3759DF) - (xi >> 1)
    y = lax.bitcast_convert_type(yi, jnp.float32)
    h = x * 0.5
    for _ in range(iters):
        y = y * (1.5 - h * y * y)
    return y


def _sc_body(featT, srcp, dstp, h1, h2, g,
             acc_sh, deg_sh, fi0, fi1, fj0, fj1, msg0, msg1,
             isrc0, isrc1, idst0, idst1, sidst0, sidst1,
             ones_v, dn_l, wb, wbg, zcol,
             sfi0, sfi1, sfj0, sfj1, sc0, sc1):
    c = lax.axis_index("c")
    s = lax.axis_index("s")
    coff = c * N_PAD
    node0 = s * NPT
    e0 = s * E_T

    zero16 = jnp.zeros((16,), jnp.float32)
    one16 = jnp.ones((16,), jnp.float32)

    def zrow(r, u):
        for j in range(8):
            wb[r, pl.ds(j * 16, 16)] = zero16
        return u
    lax.fori_loop(0, WB, zrow, 0)

    def zcol_f(i, u):
        zcol[pl.ds(i * 16, 16)] = zero16
        return u
    lax.fori_loop(0, NPT // 16, zcol_f, 0)

    for j in range(CH // 16):
        ones_v[pl.ds(j * 16, 16)] = one16

    def zacc(i, u):
        pltpu.sync_copy(wb, acc_sh.at[pl.ds(node0 + i * WB, WB)])
        return u
    lax.fori_loop(0, NWB, zacc, 0)
    pltpu.sync_copy(zcol, deg_sh.at[pl.ds(node0, NPT)])
    plsc.subcore_barrier()

    def dchunk(i, u):
        pltpu.sync_copy(dstp.at[pl.ds(e0 + i * CH, CH)], idst0)
        pltpu.sync_copy(ones_v, deg_sh.at[idst0], add=True)
        return u
    lax.fori_loop(0, NCH, dchunk, 0)
    plsc.subcore_barrier()

    pltpu.sync_copy(deg_sh.at[pl.ds(node0, NPT)], dn_l.at[pl.ds(0, NPT)])

    def dnf(i, u):
        d = jnp.maximum(dn_l[pl.ds(i * 16, 16)], 1.0)
        dn_l[pl.ds(i * 16, 16)] = _rsqrt_newton(d, 3)
        return u
    lax.fori_loop(0, NPT // 16, dnf, 0)

    def scale_rows(i):
        def wrow(r, u):
            d = dn_l[pl.ds(i * WB + r, 16)][0]
            for j in range(8):
                wb[r, pl.ds(j * 16, 16)] = wb[r, pl.ds(j * 16, 16)] * d
            return u
        lax.fori_loop(0, WB, wrow, 0)

    def pack_rows():
        def prow(r, u):
            for j in range(4):
                ai = lax.bitcast_convert_type(wb[r, pl.ds(j * 32, 16)], jnp.int32)
                bi = lax.bitcast_convert_type(wb[r, pl.ds(j * 32 + 16, 16)], jnp.int32)
                ar = (ai + jnp.int32(0x8000)) & jnp.int32(-65536)
                br = lax.shift_right_logical(bi + jnp.int32(0x8000), 16)
                wbg[r, pl.ds(j * 16, 16)] = ar | br
            return u
        lax.fori_loop(0, WB, prow, 0)

    def pchunk(i, u):
        r0 = node0 + i * WB
        pltpu.sync_copy(featT.at[pl.ds(coff + r0, WB)], wb)
        scale_rows(i)
        pack_rows()
        pltpu.sync_copy(wbg, g.at[pl.ds(coff + r0, WB)])
        return u
    lax.fori_loop(0, NWB, pchunk, 0)
    plsc.subcore_barrier()

    bufs = (
        (fi0, fj0, msg0, isrc0, idst0, sidst0, sfi0, sfj0, sc0),
        (fi1, fj1, msg1, isrc1, idst1, sidst1, sfi1, sfj1, sc1),
    )

    def issue(b, chunk):
        fi_b, fj_b, _, isrc_b, idst_b, _, sfi_b, sfj_b, _ = b
        e = e0 + chunk * CH
        pltpu.sync_copy(srcp.at[pl.ds(e, CH)], isrc_b)
        pltpu.sync_copy(dstp.at[pl.ds(e, CH)], idst_b)
        for j in range(CH // 16):
            sl = pl.ds(j * 16, 16)
            isrc_b[sl] = isrc_b[sl] + coff
            idst_b[sl] = idst_b[sl] + coff
        pltpu.async_copy(g.at[isrc_b], fj_b, sfj_b)
        pltpu.async_copy(g.at[idst_b], fi_b, sfi_b)

    def process(b, next_chunk):
        fi_b, fj_b, msg_b, isrc_b, idst_b, sidst_b, sfi_b, sfj_b, sc_b = b
        pltpu.make_async_copy(g.at[idst_b], fi_b, sfi_b).wait()
        pltpu.make_async_copy(g.at[isrc_b], fj_b, sfj_b).wait()

        mhi = jnp.int32(-65536)

        def mrow(r, v):
            for j in range(4):
                sl16 = pl.ds(j * 16, 16)
                pi = fi_b[r, sl16]
                pj = fj_b[r, sl16]
                a0 = lax.bitcast_convert_type(pi & mhi, jnp.float32)
                a1 = lax.bitcast_convert_type(lax.shift_left(pi, 16), jnp.float32)
                b0 = lax.bitcast_convert_type(pj & mhi, jnp.float32)
                b1 = lax.bitcast_convert_type(lax.shift_left(pj, 16), jnp.float32)
                for k, (a, b_) in enumerate(((a0, b0), (a1, b1))):
                    diff = a - b_
                    nd = jnp.abs(diff) + 1e-9
                    scale = nd * _rsqrt_newton(nd, 1)
                    msg_b[r, pl.ds(j * 32 + k * 16, 16)] = a - scale * diff
            return v
        lax.fori_loop(0, CH, mrow, 0)
        for j in range(CH // 16):
            sl = pl.ds(j * 16, 16)
            sidst_b[sl] = idst_b[sl] - coff
        d = pltpu.async_copy(msg_b, acc_sh.at[sidst_b], sc_b, add=True)
        e = e0 + next_chunk * CH
        pltpu.sync_copy(srcp.at[pl.ds(e, CH)], isrc_b)
        pltpu.sync_copy(dstp.at[pl.ds(e, CH)], idst_b)
        for j in range(CH // 16):
            sl = pl.ds(j * 16, 16)
            isrc_b[sl] = isrc_b[sl] + coff
            idst_b[sl] = idst_b[sl] + coff
        pltpu.async_copy(g.at[isrc_b], fj_b, sfj_b)
        pltpu.async_copy(g.at[idst_b], fi_b, sfi_b)
        d.wait()

    for step in range(KSTEPS):
        hk = h1 if step == 0 else h2
        last = step == KSTEPS - 1

        issue(bufs[0], 0)
        issue(bufs[1], 1)

        def pair(i2, u):
            process(bufs[0], 2 * i2 + 2)
            process(bufs[1], 2 * i2 + 3)
            return u
        lax.fori_loop(0, NPAIR, pair, 0)
        for b in bufs:
            fi_b, fj_b, _, isrc_b, idst_b, _, sfi_b, sfj_b, _ = b
            pltpu.make_async_copy(g.at[idst_b], fi_b, sfi_b).wait()
            pltpu.make_async_copy(g.at[isrc_b], fj_b, sfj_b).wait()
        plsc.subcore_barrier()

        def wchunk(i, u):
            r0 = node0 + i * WB
            pltpu.sync_copy(acc_sh.at[pl.ds(r0, WB)], wb)
            scale_rows(i)
            pltpu.sync_copy(wb, hk.at[pl.ds(coff + r0, WB)])
            if not last:
                scale_rows(i)
                pack_rows()
                pltpu.sync_copy(wbg, g.at[pl.ds(coff + r0, WB)])
                lax.fori_loop(0, WB, zrow, 0)
                pltpu.sync_copy(wb, acc_sh.at[pl.ds(r0, WB)])
            return u
        lax.fori_loop(0, NWB, wchunk, 0)
        if not last:
            plsc.subcore_barrier()


def _sc_propagate(featT, srcp, dstp):
    mesh = plsc.VectorSubcoreMesh(core_axis_name="c", subcore_axis_name="s")
    f32 = jnp.float32
    run = pl.kernel(
        _sc_body,
        out_type=[
            jax.ShapeDtypeStruct((2 * N_PAD, HD), f32),
            jax.ShapeDtypeStruct((2 * N_PAD, HD), f32),
            jax.ShapeDtypeStruct((2 * N_PAD, HD // 2), jnp.int32),
        ],
        mesh=mesh,
        compiler_params=pltpu.CompilerParams(use_tc_tiling_on_sc=False),
        scratch_types=[
            pltpu.VMEM_SHARED((N_PAD, HD), f32),
            pltpu.VMEM_SHARED((N_PAD,), f32),
            pltpu.VMEM((CH, HD // 2), jnp.int32),
            pltpu.VMEM((CH, HD // 2), jnp.int32),
            pltpu.VMEM((CH, HD // 2), jnp.int32),
            pltpu.VMEM((CH, HD // 2), jnp.int32),
            pltpu.VMEM((CH, HD), f32),
            pltpu.VMEM((CH, HD), f32),
            pltpu.VMEM((CH,), jnp.int32),
            pltpu.VMEM((CH,), jnp.int32),
            pltpu.VMEM((CH,), jnp.int32),
            pltpu.VMEM((CH,), jnp.int32),
            pltpu.VMEM((CH,), jnp.int32),
            pltpu.VMEM((CH,), jnp.int32),
            pltpu.VMEM((CH,), f32),
            pltpu.VMEM((NPT + 16,), f32),
            pltpu.VMEM((WB, HD), f32),
            pltpu.VMEM((WB, HD // 2), jnp.int32),
            pltpu.VMEM((NPT,), f32),
            pltpu.SemaphoreType.DMA,
            pltpu.SemaphoreType.DMA,
            pltpu.SemaphoreType.DMA,
            pltpu.SemaphoreType.DMA,
            pltpu.SemaphoreType.DMA,
            pltpu.SemaphoreType.DMA,
        ],
    )
    return run(featT, srcp, dstp)


BN = 1000


def _tc_body(al_r, b_r, fL_r, fR_r, h1a_r, h1b_r, h2a_r, h2b_r, W_r, o_r):
    a0 = al_r[0, 0]
    a1 = al_r[0, 1]
    a2 = al_r[0, 2]
    SL = a0 * fL_r[...] + a1 * h1a_r[...] + a2 * h2a_r[...]
    SR = a0 * fR_r[...] + a1 * h1b_r[...] + a2 * h2b_r[...]
    wl = W_r[:, :HD]
    wr = W_r[:, HD:]
    dn = (((1,), (1,)), ((), ()))
    acc = lax.dot_general(SL, wl, dn, precision=lax.Precision.HIGHEST,
                          preferred_element_type=jnp.float32)
    acc = acc + lax.dot_general(SR, wr, dn, precision=lax.Precision.HIGHEST,
                                preferred_element_type=jnp.float32)
    o_r[...] = acc + (KSTEPS + 1) * b_r[...]


def _tc_combine(alpha2, b2, fL, fR, h1a, h1b, h2a, h2b, W):
    f32 = jnp.float32
    half = pl.BlockSpec((BN, HD), lambda i: (i, 0))
    fixed = lambda shape: pl.BlockSpec(shape, lambda i: (0, 0))
    return pl.pallas_call(
        _tc_body,
        grid=(N // BN,),
        in_specs=[
            fixed((1, 3)),
            fixed((1, D)),
            half, half, half, half, half, half,
            fixed((D, D)),
        ],
        out_specs=pl.BlockSpec((BN, D), lambda i: (i, 0)),
        out_shape=jax.ShapeDtypeStruct((N, D), f32),
    )(alpha2, b2, fL, fR, h1a, h1b, h2a, h2b, W)


def kernel(feat, edge_index, W, b, alpha):
    f32 = jnp.float32
    src = edge_index[0].astype(jnp.int32)
    dst = edge_index[1].astype(jnp.int32)
    npad = E_PAD + 2 * CH - E
    pad = (N + (jnp.arange(npad, dtype=jnp.int32) % NPADROWS)).astype(jnp.int32)
    srcp = jnp.concatenate([src, pad])
    dstp = jnp.concatenate([dst, pad])

    featp = jnp.pad(feat.astype(f32), ((0, N_PAD - N), (0, 0)))
    featT = jnp.transpose(featp.reshape(N_PAD, 2, HD), (1, 0, 2)).reshape(2 * N_PAD, HD)

    h1, h2, _ = _sc_propagate(featT, srcp, dstp)

    h1a, h1b = h1[:N], h1[N_PAD:N_PAD + N]
    h2a, h2b = h2[:N], h2[N_PAD:N_PAD + N]
    fL, fR = feat[:, :HD], feat[:, HD:]
    alpha2 = alpha.reshape(1, 3).astype(f32)
    b2 = b.reshape(1, D).astype(f32)

    return _tc_combine(alpha2, b2, fL, fR, h1a, h1b, h2a, h2b, W.astype(f32))


# speedup vs baseline: 2.3883x; 2.3883x over previous
import jax
import jax.numpy as jnp
from jax import lax
from jax.experimental import pallas as pl
from jax.experimental.pallas import tpu as pltpu
from jax.experimental.pallas import tpu_sc as plsc

N = 10000
E = 160000
D = 256
HD = 128
KSTEPS = 2
P = 2.5

NTILES = 16
N_PAD = 10240
NPT = N_PAD // NTILES
E_T = 10240
E_PAD = E_T * NTILES
CH = 64
NCH = E_T // CH
NPAIR = NCH // 2
PADCH = 4
WB = 16
NWB = NPT // WB
NPADROWS = N_PAD - N


def _rsqrt_newton(x, iters):
    xi = lax.bitcast_convert_type(x, jnp.int32)
    yi = jnp.int32(0x5F3759DF) - (xi >> 1)
    y = lax.bitcast_convert_type(yi, jnp.float32)
    h = x * 0.5
    for _ in range(iters):
        y = y * (1.5 - h * y * y)
    return y


def _sc_body(featT, epk, h1, h2, g,
             acc_sh, deg_sh, fi0, fi1, fj0, fj1,
             idx0, idx1, gsrc0, gsrc1, gdst0, gdst1, sidst0, sidst1,
             ones_v, dn_l, wb, zcol,
             sfi0, sfi1, sfj0, sfj1, sc0, sc1, si0, si1):
    c = lax.axis_index("c")
    s = lax.axis_index("s")
    coff = c * N_PAD
    node0 = s * NPT
    ci0 = s * NCH

    zero16 = jnp.zeros((16,), jnp.float32)
    one16 = jnp.ones((16,), jnp.float32)

    def zrow(r, u):
        for j in range(8):
            wb[r, pl.ds(j * 16, 16)] = zero16
        return u
    lax.fori_loop(0, WB, zrow, 0)

    def zcol_f(i, u):
        zcol[pl.ds(i * 16, 16)] = zero16
        return u
    lax.fori_loop(0, NPT // 16, zcol_f, 0)

    for j in range(CH // 16):
        ones_v[pl.ds(j * 16, 16)] = one16

    def zacc(i, u):
        pltpu.sync_copy(wb, acc_sh.at[pl.ds(node0 + i * WB, WB)])
        return u
    lax.fori_loop(0, NWB, zacc, 0)
    pltpu.sync_copy(zcol, deg_sh.at[pl.ds(node0, NPT)])
    plsc.subcore_barrier()

    def dchunk(i, u):
        pltpu.sync_copy(epk.at[pl.ds((ci0 + i) * 2 * CH + CH, CH)], sidst0)
        pltpu.sync_copy(ones_v, deg_sh.at[sidst0], add=True)
        return u
    lax.fori_loop(0, NCH, dchunk, 0)
    plsc.subcore_barrier()

    pltpu.sync_copy(deg_sh.at[pl.ds(node0, NPT)], dn_l.at[pl.ds(0, NPT)])

    def dnf(i, u):
        d = jnp.maximum(dn_l[pl.ds(i * 16, 16)], 1.0)
        dn_l[pl.ds(i * 16, 16)] = _rsqrt_newton(d, 3)
        return u
    lax.fori_loop(0, NPT // 16, dnf, 0)

    def scale_rows(i):
        def wrow(r, u):
            d = dn_l[pl.ds(i * WB + r, 16)][0]
            for j in range(8):
                wb[r, pl.ds(j * 16, 16)] = wb[r, pl.ds(j * 16, 16)] * d
            return u
        lax.fori_loop(0, WB, wrow, 0)

    def pchunk(i, u):
        r0 = node0 + i * WB
        pltpu.sync_copy(featT.at[pl.ds(coff + r0, WB)], wb)
        scale_rows(i)
        pltpu.sync_copy(wb, g.at[pl.ds(coff + r0, WB)])
        return u
    lax.fori_loop(0, NWB, pchunk, 0)
    plsc.subcore_barrier()

    bufs = (
        (fi0, fj0, idx0, gsrc0, gdst0, sidst0, sfi0, sfj0, sc0, si0),
        (fi1, fj1, idx1, gsrc1, gdst1, sidst1, sfi1, sfj1, sc1, si1),
    )

    def idx_slice(chunk):
        return epk.at[pl.ds((ci0 + chunk) * 2 * CH, 2 * CH)]

    def adjust(b):
        _, _, idx_b, gsrc_b, gdst_b, _, _, _, _, _ = b
        for j in range(CH // 16):
            sl = pl.ds(j * 16, 16)
            gsrc_b[sl] = idx_b[sl] + coff
            gdst_b[sl] = idx_b[pl.ds(CH + j * 16, 16)] + coff

    def gathers(b):
        fi_b, fj_b, _, gsrc_b, gdst_b, _, sfi_b, sfj_b, _, _ = b
        pltpu.async_copy(g.at[gsrc_b], fj_b, sfj_b)
        pltpu.async_copy(g.at[gdst_b], fi_b, sfi_b)

    def process(b, chunk):
        fi_b, fj_b, idx_b, gsrc_b, gdst_b, sidst_b, sfi_b, sfj_b, sc_b, si_b = b
        pltpu.make_async_copy(g.at[gdst_b], fi_b, sfi_b).wait()
        pltpu.make_async_copy(g.at[gsrc_b], fj_b, sfj_b).wait()

        def mrow(r, v):
            for j in range(8):
                sl = pl.ds(j * 16, 16)
                a = fi_b[r, sl]
                b_ = fj_b[r, sl]
                diff = a - b_
                nd = jnp.abs(diff) + 1e-9
                scale = nd * _rsqrt_newton(nd, 1)
                fi_b[r, sl] = a - scale * diff
            return v
        lax.fori_loop(0, CH, mrow, 0)

        for j in range(CH // 16):
            sl = pl.ds(j * 16, 16)
            sidst_b[sl] = gdst_b[sl] - coff
        d = pltpu.async_copy(fi_b, acc_sh.at[sidst_b], sc_b, add=True)

        pltpu.make_async_copy(idx_slice(chunk + 2), idx_b, si_b).wait()
        adjust(b)
        pltpu.async_copy(g.at[gsrc_b], fj_b, sfj_b)
        d.wait()
        pltpu.async_copy(g.at[gdst_b], fi_b, sfi_b)
        pltpu.async_copy(idx_slice(chunk + 4), idx_b, si_b)

    for step in range(KSTEPS):
        hk = h1 if step == 0 else h2
        last = step == KSTEPS - 1

        for k, b in enumerate(bufs):
            pltpu.sync_copy(idx_slice(k), b[2])
            adjust(b)
            gathers(b)
        pltpu.async_copy(idx_slice(2), idx0, si0)
        pltpu.async_copy(idx_slice(3), idx1, si1)

        def pairf(i2, u):
            process(bufs[0], 2 * i2)
            process(bufs[1], 2 * i2 + 1)
            return u
        lax.fori_loop(0, NPAIR, pairf, 0)

        for k, b in enumerate(bufs):
            fi_b, fj_b, idx_b, gsrc_b, gdst_b, _, sfi_b, sfj_b, _, si_b = b
            pltpu.make_async_copy(g.at[gdst_b], fi_b, sfi_b).wait()
            pltpu.make_async_copy(g.at[gsrc_b], fj_b, sfj_b).wait()
            pltpu.make_async_copy(idx_slice(NCH + 2 + k), idx_b, si_b).wait()
        plsc.subcore_barrier()

        def wchunk(i, u):
            r0 = node0 + i * WB
            pltpu.sync_copy(acc_sh.at[pl.ds(r0, WB)], wb)
            scale_rows(i)
            pltpu.sync_copy(wb, hk.at[pl.ds(coff + r0, WB)])
            if not last:
                scale_rows(i)
                pltpu.sync_copy(wb, g.at[pl.ds(coff + r0, WB)])
                lax.fori_loop(0, WB, zrow, 0)
                pltpu.sync_copy(wb, acc_sh.at[pl.ds(r0, WB)])
            return u
        lax.fori_loop(0, NWB, wchunk, 0)
        if not last:
            plsc.subcore_barrier()


def _sc_propagate(featT, epk):
    mesh = plsc.VectorSubcoreMesh(core_axis_name="c", subcore_axis_name="s")
    f32 = jnp.float32
    run = pl.kernel(
        _sc_body,
        out_type=[
            jax.ShapeDtypeStruct((2 * N_PAD, HD), f32),
            jax.ShapeDtypeStruct((2 * N_PAD, HD), f32),
            jax.ShapeDtypeStruct((2 * N_PAD, HD), f32),
        ],
        mesh=mesh,
        scratch_types=[
            pltpu.VMEM_SHARED((N_PAD, HD), f32),
            pltpu.VMEM_SHARED((N_PAD,), f32),
            pltpu.VMEM((CH, HD), f32),
            pltpu.VMEM((CH, HD), f32),
            pltpu.VMEM((CH, HD), f32),
            pltpu.VMEM((CH, HD), f32),
            pltpu.VMEM((2 * CH,), jnp.int32),
            pltpu.VMEM((2 * CH,), jnp.int32),
            pltpu.VMEM((CH,), jnp.int32),
            pltpu.VMEM((CH,), jnp.int32),
            pltpu.VMEM((CH,), jnp.int32),
            pltpu.VMEM((CH,), jnp.int32),
            pltpu.VMEM((CH,), jnp.int32),
            pltpu.VMEM((CH,), jnp.int32),
            pltpu.VMEM((CH,), f32),
            pltpu.VMEM((NPT + 16,), f32),
            pltpu.VMEM((WB, HD), f32),
            pltpu.VMEM((NPT,), f32),
            pltpu.SemaphoreType.DMA,
            pltpu.SemaphoreType.DMA,
            pltpu.SemaphoreType.DMA,
            pltpu.SemaphoreType.DMA,
            pltpu.SemaphoreType.DMA,
            pltpu.SemaphoreType.DMA,
            pltpu.SemaphoreType.DMA,
            pltpu.SemaphoreType.DMA,
        ],
    )
    return run(featT, epk)


BN = 1000


def _tc_body(al_r, b_r, fL_r, fR_r, h1a_r, h1b_r, h2a_r, h2b_r, W_r, o_r):
    a0 = al_r[0, 0]
    a1 = al_r[0, 1]
    a2 = al_r[0, 2]
    SL = a0 * fL_r[...] + a1 * h1a_r[...] + a2 * h2a_r[...]
    SR = a0 * fR_r[...] + a1 * h1b_r[...] + a2 * h2b_r[...]
    wl = W_r[:, :HD]
    wr = W_r[:, HD:]
    dn = (((1,), (1,)), ((), ()))
    acc = lax.dot_general(SL, wl, dn, precision=lax.Precision.HIGHEST,
                          preferred_element_type=jnp.float32)
    acc = acc + lax.dot_general(SR, wr, dn, precision=lax.Precision.HIGHEST,
                                preferred_element_type=jnp.float32)
    o_r[...] = acc + (KSTEPS + 1) * b_r[...]


def _tc_combine(alpha2, b2, fL, fR, h1a, h1b, h2a, h2b, W):
    f32 = jnp.float32
    half = pl.BlockSpec((BN, HD), lambda i: (i, 0))
    fixed = lambda shape: pl.BlockSpec(shape, lambda i: (0, 0))
    return pl.pallas_call(
        _tc_body,
        grid=(N // BN,),
        in_specs=[
            fixed((1, 3)),
            fixed((1, D)),
            half, half, half, half, half, half,
            fixed((D, D)),
        ],
        out_specs=pl.BlockSpec((BN, D), lambda i: (i, 0)),
        out_shape=jax.ShapeDtypeStruct((N, D), f32),
    )(alpha2, b2, fL, fR, h1a, h1b, h2a, h2b, W)


def kernel(feat, edge_index, W, b, alpha):
    f32 = jnp.float32
    src = edge_index[0].astype(jnp.int32)
    dst = edge_index[1].astype(jnp.int32)
    npad = E_PAD + PADCH * CH - E
    pad = (N + (jnp.arange(npad, dtype=jnp.int32) % NPADROWS)).astype(jnp.int32)
    srcp = jnp.concatenate([src, pad])
    dstp = jnp.concatenate([dst, pad])
    epk = jnp.stack([srcp.reshape(-1, CH), dstp.reshape(-1, CH)], axis=1).reshape(-1)

    featp = jnp.pad(feat.astype(f32), ((0, N_PAD - N), (0, 0)))
    featT = jnp.transpose(featp.reshape(N_PAD, 2, HD), (1, 0, 2)).reshape(2 * N_PAD, HD)

    h1, h2, _ = _sc_propagate(featT, epk)

    h1a, h1b = h1[:N], h1[N_PAD:N_PAD + N]
    h2a, h2b = h2[:N], h2[N_PAD:N_PAD + N]
    fL, fR = feat[:, :HD], feat[:, HD:]
    alpha2 = alpha.reshape(1, 3).astype(f32)
    b2 = b.reshape(1, D).astype(f32)

    return _tc_combine(alpha2, b2, fL, fR, h1a, h1b, h2a, h2b, W.astype(f32))


# speedup vs baseline: 2.5034x; 1.0482x over previous
import jax
import jax.numpy as jnp
from jax import lax
from jax.experimental import pallas as pl
from jax.experimental.pallas import tpu as pltpu
from jax.experimental.pallas import tpu_sc as plsc

N = 10000
E = 160000
D = 256
HD = 128
KSTEPS = 2
P = 2.5

NTILES = 16
N_PAD = 10240
NPT = N_PAD // NTILES
E_T = 10240
E_PAD = E_T * NTILES
CH = 64
NCH = E_T // CH
NPAIR = NCH // 2
PADCH = 4
WB = 16
NWB = NPT // WB
NPADROWS = N_PAD - N


def _rsqrt_newton(x, iters):
    xi = lax.bitcast_convert_type(x, jnp.int32)
    yi = jnp.int32(0x5F3759DF) - (xi >> 1)
    y = lax.bitcast_convert_type(yi, jnp.float32)
    h = x * 0.5
    for _ in range(iters):
        y = y * (1.5 - h * y * y)
    return y


def _sc_body(featT, epk, h1, h2, g,
             acc_sh, deg_sh, fi0, fi1, fj0, fj1,
             idx0, idx1, gsrc0, gsrc1, gdst0, gdst1, sidst0, sidst1,
             ones_v, dn_l, wb, zcol,
             sfi0, sfi1, sfj0, sfj1, sc0, sc1, si0, si1):
    c = lax.axis_index("c")
    s = lax.axis_index("s")
    coff = c * N_PAD
    node0 = s * NPT
    ci0 = s * NCH

    zero16 = jnp.zeros((16,), jnp.float32)
    one16 = jnp.ones((16,), jnp.float32)

    def zrow(r, u):
        for j in range(8):
            wb[r, pl.ds(j * 16, 16)] = zero16
        return u
    lax.fori_loop(0, WB, zrow, 0)

    def zcol_f(i, u):
        zcol[pl.ds(i * 16, 16)] = zero16
        return u
    lax.fori_loop(0, NPT // 16, zcol_f, 0)

    for j in range(CH // 16):
        ones_v[pl.ds(j * 16, 16)] = one16

    def zacc(i, u):
        pltpu.sync_copy(wb, acc_sh.at[pl.ds(node0 + i * WB, WB)])
        return u
    lax.fori_loop(0, NWB, zacc, 0)
    pltpu.sync_copy(zcol, deg_sh.at[pl.ds(node0, NPT)])
    plsc.subcore_barrier()

    def dst_slice(chunk):
        return epk.at[pl.ds((ci0 + chunk) * 2 * CH + CH, CH)]

    pltpu.async_copy(dst_slice(0), sidst0, si0)
    pltpu.async_copy(dst_slice(1), sidst1, si1)

    def dpair(i2, u):
        pltpu.make_async_copy(dst_slice(2 * i2), sidst0, si0).wait()
        d0 = pltpu.async_copy(ones_v, deg_sh.at[sidst0], sc0, add=True)
        pltpu.make_async_copy(dst_slice(2 * i2 + 1), sidst1, si1).wait()
        d1 = pltpu.async_copy(ones_v, deg_sh.at[sidst1], sc1, add=True)
        d0.wait()
        pltpu.async_copy(dst_slice(2 * i2 + 2), sidst0, si0)
        d1.wait()
        pltpu.async_copy(dst_slice(2 * i2 + 3), sidst1, si1)
        return u
    lax.fori_loop(0, NPAIR, dpair, 0)
    pltpu.make_async_copy(dst_slice(NCH), sidst0, si0).wait()
    pltpu.make_async_copy(dst_slice(NCH + 1), sidst1, si1).wait()
    plsc.subcore_barrier()

    pltpu.sync_copy(deg_sh.at[pl.ds(node0, NPT)], dn_l.at[pl.ds(0, NPT)])

    def dnf(i, u):
        d = jnp.maximum(dn_l[pl.ds(i * 16, 16)], 1.0)
        dn_l[pl.ds(i * 16, 16)] = _rsqrt_newton(d, 3)
        return u
    lax.fori_loop(0, NPT // 16, dnf, 0)

    def scale_rows(i):
        def wrow(r, u):
            d = dn_l[pl.ds(i * WB + r, 16)][0]
            for j in range(8):
                wb[r, pl.ds(j * 16, 16)] = wb[r, pl.ds(j * 16, 16)] * d
            return u
        lax.fori_loop(0, WB, wrow, 0)

    def pchunk(i, u):
        r0 = node0 + i * WB
        pltpu.sync_copy(featT.at[pl.ds(coff + r0, WB)], wb)
        scale_rows(i)
        pltpu.sync_copy(wb, g.at[pl.ds(coff + r0, WB)])
        return u
    lax.fori_loop(0, NWB, pchunk, 0)
    plsc.subcore_barrier()

    bufs = (
        (fi0, fj0, idx0, gsrc0, gdst0, sidst0, sfi0, sfj0, sc0, si0),
        (fi1, fj1, idx1, gsrc1, gdst1, sidst1, sfi1, sfj1, sc1, si1),
    )

    def idx_slice(chunk):
        return epk.at[pl.ds((ci0 + chunk) * 2 * CH, 2 * CH)]

    def adjust(b):
        _, _, idx_b, gsrc_b, gdst_b, _, _, _, _, _ = b
        for j in range(CH // 16):
            sl = pl.ds(j * 16, 16)
            gsrc_b[sl] = idx_b[sl] + coff
            gdst_b[sl] = idx_b[pl.ds(CH + j * 16, 16)] + coff

    def gathers(b):
        fi_b, fj_b, _, gsrc_b, gdst_b, _, sfi_b, sfj_b, _, _ = b
        pltpu.async_copy(g.at[gsrc_b], fj_b, sfj_b)
        pltpu.async_copy(g.at[gdst_b], fi_b, sfi_b)

    def process(b, chunk):
        fi_b, fj_b, idx_b, gsrc_b, gdst_b, sidst_b, sfi_b, sfj_b, sc_b, si_b = b
        pltpu.make_async_copy(g.at[gdst_b], fi_b, sfi_b).wait()
        pltpu.make_async_copy(g.at[gsrc_b], fj_b, sfj_b).wait()

        def mrow(r, v):
            for j in range(8):
                sl = pl.ds(j * 16, 16)
                a = fi_b[r, sl]
                b_ = fj_b[r, sl]
                diff = a - b_
                nd = jnp.abs(diff) + 1e-9
                scale = nd * _rsqrt_newton(nd, 1)
                fi_b[r, sl] = a - scale * diff
            return v
        lax.fori_loop(0, CH, mrow, 0)

        for j in range(CH // 16):
            sl = pl.ds(j * 16, 16)
            sidst_b[sl] = gdst_b[sl] - coff
        d = pltpu.async_copy(fi_b, acc_sh.at[sidst_b], sc_b, add=True)

        pltpu.make_async_copy(idx_slice(chunk + 2), idx_b, si_b).wait()
        adjust(b)
        pltpu.async_copy(g.at[gsrc_b], fj_b, sfj_b)
        d.wait()
        pltpu.async_copy(g.at[gdst_b], fi_b, sfi_b)
        pltpu.async_copy(idx_slice(chunk + 4), idx_b, si_b)

    for step in range(KSTEPS):
        hk = h1 if step == 0 else h2
        last = step == KSTEPS - 1

        for k, b in enumerate(bufs):
            pltpu.sync_copy(idx_slice(k), b[2])
            adjust(b)
            gathers(b)
        pltpu.async_copy(idx_slice(2), idx0, si0)
        pltpu.async_copy(idx_slice(3), idx1, si1)

        def pairf(i2, u):
            process(bufs[0], 2 * i2)
            process(bufs[1], 2 * i2 + 1)
            return u
        lax.fori_loop(0, NPAIR, pairf, 0)

        for k, b in enumerate(bufs):
            fi_b, fj_b, idx_b, gsrc_b, gdst_b, _, sfi_b, sfj_b, _, si_b = b
            pltpu.make_async_copy(g.at[gdst_b], fi_b, sfi_b).wait()
            pltpu.make_async_copy(g.at[gsrc_b], fj_b, sfj_b).wait()
            pltpu.make_async_copy(idx_slice(NCH + 2 + k), idx_b, si_b).wait()
        plsc.subcore_barrier()

        def wchunk(i, u):
            r0 = node0 + i * WB
            pltpu.sync_copy(acc_sh.at[pl.ds(r0, WB)], wb)
            scale_rows(i)
            pltpu.sync_copy(wb, hk.at[pl.ds(coff + r0, WB)])
            if not last:
                scale_rows(i)
                pltpu.sync_copy(wb, g.at[pl.ds(coff + r0, WB)])
                lax.fori_loop(0, WB, zrow, 0)
                pltpu.sync_copy(wb, acc_sh.at[pl.ds(r0, WB)])
            return u
        lax.fori_loop(0, NWB, wchunk, 0)
        if not last:
            plsc.subcore_barrier()


def _sc_propagate(featT, epk):
    mesh = plsc.VectorSubcoreMesh(core_axis_name="c", subcore_axis_name="s")
    f32 = jnp.float32
    run = pl.kernel(
        _sc_body,
        out_type=[
            jax.ShapeDtypeStruct((2 * N_PAD, HD), f32),
            jax.ShapeDtypeStruct((2 * N_PAD, HD), f32),
            jax.ShapeDtypeStruct((2 * N_PAD, HD), f32),
        ],
        mesh=mesh,
        scratch_types=[
            pltpu.VMEM_SHARED((N_PAD, HD), f32),
            pltpu.VMEM_SHARED((N_PAD,), f32),
            pltpu.VMEM((CH, HD), f32),
            pltpu.VMEM((CH, HD), f32),
            pltpu.VMEM((CH, HD), f32),
            pltpu.VMEM((CH, HD), f32),
            pltpu.VMEM((2 * CH,), jnp.int32),
            pltpu.VMEM((2 * CH,), jnp.int32),
            pltpu.VMEM((CH,), jnp.int32),
            pltpu.VMEM((CH,), jnp.int32),
            pltpu.VMEM((CH,), jnp.int32),
            pltpu.VMEM((CH,), jnp.int32),
            pltpu.VMEM((CH,), jnp.int32),
            pltpu.VMEM((CH,), jnp.int32),
            pltpu.VMEM((CH,), f32),
            pltpu.VMEM((NPT + 16,), f32),
            pltpu.VMEM((WB, HD), f32),
            pltpu.VMEM((NPT,), f32),
            pltpu.SemaphoreType.DMA,
            pltpu.SemaphoreType.DMA,
            pltpu.SemaphoreType.DMA,
            pltpu.SemaphoreType.DMA,
            pltpu.SemaphoreType.DMA,
            pltpu.SemaphoreType.DMA,
            pltpu.SemaphoreType.DMA,
            pltpu.SemaphoreType.DMA,
        ],
    )
    return run(featT, epk)


BN = 1000


def _tc_body(al_r, b_r, fL_r, fR_r, h1a_r, h1b_r, h2a_r, h2b_r, W_r, o_r):
    a0 = al_r[0, 0]
    a1 = al_r[0, 1]
    a2 = al_r[0, 2]
    SL = a0 * fL_r[...] + a1 * h1a_r[...] + a2 * h2a_r[...]
    SR = a0 * fR_r[...] + a1 * h1b_r[...] + a2 * h2b_r[...]
    wl = W_r[:, :HD]
    wr = W_r[:, HD:]
    dn = (((1,), (1,)), ((), ()))
    acc = lax.dot_general(SL, wl, dn, precision=lax.Precision.HIGHEST,
                          preferred_element_type=jnp.float32)
    acc = acc + lax.dot_general(SR, wr, dn, precision=lax.Precision.HIGHEST,
                                preferred_element_type=jnp.float32)
    o_r[...] = acc + (KSTEPS + 1) * b_r[...]


def _tc_combine(alpha2, b2, fL, fR, h1a, h1b, h2a, h2b, W):
    f32 = jnp.float32
    half = pl.BlockSpec((BN, HD), lambda i: (i, 0))
    fixed = lambda shape: pl.BlockSpec(shape, lambda i: (0, 0))
    return pl.pallas_call(
        _tc_body,
        grid=(N // BN,),
        in_specs=[
            fixed((1, 3)),
            fixed((1, D)),
            half, half, half, half, half, half,
            fixed((D, D)),
        ],
        out_specs=pl.BlockSpec((BN, D), lambda i: (i, 0)),
        out_shape=jax.ShapeDtypeStruct((N, D), f32),
    )(alpha2, b2, fL, fR, h1a, h1b, h2a, h2b, W)


def kernel(feat, edge_index, W, b, alpha):
    f32 = jnp.float32
    src = edge_index[0].astype(jnp.int32)
    dst = edge_index[1].astype(jnp.int32)
    npad = E_PAD + PADCH * CH - E
    pad = (N + (jnp.arange(npad, dtype=jnp.int32) % NPADROWS)).astype(jnp.int32)
    srcp = jnp.concatenate([src, pad])
    dstp = jnp.concatenate([dst, pad])
    epk = jnp.stack([srcp.reshape(-1, CH), dstp.reshape(-1, CH)], axis=1).reshape(-1)

    featp = jnp.pad(feat.astype(f32), ((0, N_PAD - N), (0, 0)))
    featT = jnp.transpose(featp.reshape(N_PAD, 2, HD), (1, 0, 2)).reshape(2 * N_PAD, HD)

    h1, h2, _ = _sc_propagate(featT, epk)

    h1a, h1b = h1[:N], h1[N_PAD:N_PAD + N]
    h2a, h2b = h2[:N], h2[N_PAD:N_PAD + N]
    fL, fR = feat[:, :HD], feat[:, HD:]
    alpha2 = alpha.reshape(1, 3).astype(f32)
    b2 = b.reshape(1, D).astype(f32)

    return _tc_combine(alpha2, b2, fL, fR, h1a, h1b, h2a, h2b, W.astype(f32))


# speedup vs baseline: 2.5632x; 1.0239x over previous
import jax
import jax.numpy as jnp
from jax import lax
from jax.experimental import pallas as pl
from jax.experimental.pallas import tpu as pltpu
from jax.experimental.pallas import tpu_sc as plsc

N = 10000
E = 160000
D = 256
HD = 128
KSTEPS = 2
P = 2.5

NTILES = 16
N_PAD = 10240
NPT = N_PAD // NTILES
E_T = 10240
E_PAD = E_T * NTILES
CH = 64
NCH = E_T // CH
NPAIR = NCH // 2
PADCH = 4
WB = 8
NWB = NPT // WB
NPADROWS = N_PAD - N


def _rsqrt_newton(x, iters):
    xi = lax.bitcast_convert_type(x, jnp.int32)
    yi = jnp.int32(0x5F3759DF) - (xi >> 1)
    y = lax.bitcast_convert_type(yi, jnp.float32)
    h = x * 0.5
    for _ in range(iters):
        y = y * (1.5 - h * y * y)
    return y


def _sc_body(featT, epk, h2, g, dnv,
             acc_sh, deg_sh, fi0, fi1, fj0, fj1,
             idx0, idx1, gsrc0, gsrc1, gdst0, gdst1, sidst0, sidst1,
             ones_v, dn_l, wb0, wb1, zbuf, zcol,
             sfi0, sfi1, sfj0, sfj1, sc0, sc1, si0, si1):
    c = lax.axis_index("c")
    s = lax.axis_index("s")
    coff = c * N_PAD
    node0 = s * NPT
    ci0 = s * NCH

    zero16 = jnp.zeros((16,), jnp.float32)
    one16 = jnp.ones((16,), jnp.float32)

    def rows(i):
        return pl.ds(node0 + jnp.minimum(i, NWB - 1) * WB, WB)

    def hrows(i):
        return pl.ds(coff + node0 + jnp.minimum(i, NWB - 1) * WB, WB)

    def zrow(r, u):
        for j in range(8):
            zbuf[r, pl.ds(j * 16, 16)] = zero16
        return u
    lax.fori_loop(0, WB, zrow, 0)

    def zcol_f(i, u):
        zcol[pl.ds(i * 16, 16)] = zero16
        return u
    lax.fori_loop(0, NPT // 16, zcol_f, 0)

    for j in range(CH // 16):
        ones_v[pl.ds(j * 16, 16)] = one16

    def zacc(i, u):
        pltpu.sync_copy(zbuf, acc_sh.at[rows(i)])
        return u
    lax.fori_loop(0, NWB, zacc, 0)
    pltpu.sync_copy(zcol, deg_sh.at[pl.ds(node0, NPT)])
    plsc.subcore_barrier()

    def dst_slice(chunk):
        return epk.at[pl.ds((ci0 + chunk) * 2 * CH + CH, CH)]

    pltpu.async_copy(dst_slice(0), sidst0, si0)
    pltpu.async_copy(dst_slice(1), sidst1, si1)

    def dpair(i2, u):
        pltpu.make_async_copy(dst_slice(2 * i2), sidst0, si0).wait()
        d0 = pltpu.async_copy(ones_v, deg_sh.at[sidst0], sc0, add=True)
        pltpu.make_async_copy(dst_slice(2 * i2 + 1), sidst1, si1).wait()
        d1 = pltpu.async_copy(ones_v, deg_sh.at[sidst1], sc1, add=True)
        d0.wait()
        pltpu.async_copy(dst_slice(2 * i2 + 2), sidst0, si0)
        d1.wait()
        pltpu.async_copy(dst_slice(2 * i2 + 3), sidst1, si1)
        return u
    lax.fori_loop(0, NPAIR, dpair, 0)
    pltpu.make_async_copy(dst_slice(NCH), sidst0, si0).wait()
    pltpu.make_async_copy(dst_slice(NCH + 1), sidst1, si1).wait()
    plsc.subcore_barrier()

    pltpu.sync_copy(deg_sh.at[pl.ds(node0, NPT)], dn_l.at[pl.ds(0, NPT)])

    def dnf(i, u):
        d = jnp.maximum(dn_l[pl.ds(i * 16, 16)], 1.0)
        dn_l[pl.ds(i * 16, 16)] = _rsqrt_newton(d, 3)
        return u
    lax.fori_loop(0, NPT // 16, dnf, 0)

    @pl.when(c == 0)
    def _():
        pltpu.sync_copy(dn_l.at[pl.ds(0, NPT)], dnv.at[pl.ds(node0, NPT)])

    def scale_rows(i, w, sq):
        def wrow(r, u):
            d = dn_l[pl.ds(jnp.minimum(i, NWB - 1) * WB + r, 16)][0]
            if sq:
                d = d * d
            for j in range(8):
                w[r, pl.ds(j * 16, 16)] = w[r, pl.ds(j * 16, 16)] * d
            return u
        lax.fori_loop(0, WB, wrow, 0)

    wbufs = ((wb0, sfi0, sc0, si0), (wb1, sfi1, sc1, si1))

    def pingpong(src_at, dst_at, sq, zero_acc):
        for k, (w, sr, sw, sz) in enumerate(wbufs):
            pltpu.async_copy(src_at(k), w, sr)

        def ppair(i2, u):
            for k, (w, sr, sw, sz) in enumerate(wbufs):
                i = 2 * i2 + k
                pltpu.make_async_copy(src_at(i), w, sr).wait()
                scale_rows(i, w, sq)
                dw = pltpu.async_copy(w, dst_at(i), sw)
                if zero_acc:
                    dz = pltpu.async_copy(zbuf, acc_sh.at[rows(i)], sz)
                dw.wait()
                if zero_acc:
                    dz.wait()
                pltpu.async_copy(src_at(i + 2), w, sr)
            return u
        lax.fori_loop(0, NWB // 2, ppair, 0)
        for k, (w, sr, sw, sz) in enumerate(wbufs):
            pltpu.make_async_copy(src_at(0), w, sr).wait()

    pingpong(lambda i: featT.at[hrows(i)], lambda i: g.at[hrows(i)],
             sq=False, zero_acc=False)
    plsc.subcore_barrier()

    bufs = (
        (fi0, fj0, idx0, gsrc0, gdst0, sidst0, sfi0, sfj0, sc0, si0),
        (fi1, fj1, idx1, gsrc1, gdst1, sidst1, sfi1, sfj1, sc1, si1),
    )

    def idx_slice(chunk):
        return epk.at[pl.ds((ci0 + chunk) * 2 * CH, 2 * CH)]

    def adjust(b):
        _, _, idx_b, gsrc_b, gdst_b, _, _, _, _, _ = b
        for j in range(CH // 16):
            sl = pl.ds(j * 16, 16)
            gsrc_b[sl] = idx_b[sl] + coff
            gdst_b[sl] = idx_b[pl.ds(CH + j * 16, 16)] + coff

    def gathers(b):
        fi_b, fj_b, _, gsrc_b, gdst_b, _, sfi_b, sfj_b, _, _ = b
        pltpu.async_copy(g.at[gsrc_b], fj_b, sfj_b)
        pltpu.async_copy(g.at[gdst_b], fi_b, sfi_b)

    def process(b, chunk):
        fi_b, fj_b, idx_b, gsrc_b, gdst_b, sidst_b, sfi_b, sfj_b, sc_b, si_b = b
        pltpu.make_async_copy(g.at[gdst_b], fi_b, sfi_b).wait()
        pltpu.make_async_copy(g.at[gsrc_b], fj_b, sfj_b).wait()

        def mrow(r, v):
            for j in range(8):
                sl = pl.ds(j * 16, 16)
                a = fi_b[r, sl]
                b_ = fj_b[r, sl]
                diff = a - b_
                nd = jnp.abs(diff) + 1e-9
                scale = nd * _rsqrt_newton(nd, 1)
                fi_b[r, sl] = a - scale * diff
            return v
        lax.fori_loop(0, CH, mrow, 0)

        for j in range(CH // 16):
            sl = pl.ds(j * 16, 16)
            sidst_b[sl] = gdst_b[sl] - coff
        d = pltpu.async_copy(fi_b, acc_sh.at[sidst_b], sc_b, add=True)

        pltpu.make_async_copy(idx_slice(chunk + 2), idx_b, si_b).wait()
        adjust(b)
        pltpu.async_copy(g.at[gsrc_b], fj_b, sfj_b)
        d.wait()
        pltpu.async_copy(g.at[gdst_b], fi_b, sfi_b)
        pltpu.async_copy(idx_slice(chunk + 4), idx_b, si_b)

    for step in range(KSTEPS):
        last = step == KSTEPS - 1

        for k, b in enumerate(bufs):
            pltpu.sync_copy(idx_slice(k), b[2])
            adjust(b)
            gathers(b)
        pltpu.async_copy(idx_slice(2), idx0, si0)
        pltpu.async_copy(idx_slice(3), idx1, si1)

        def pairf(i2, u):
            process(bufs[0], 2 * i2)
            process(bufs[1], 2 * i2 + 1)
            return u
        lax.fori_loop(0, NPAIR, pairf, 0)

        for k, b in enumerate(bufs):
            fi_b, fj_b, idx_b, gsrc_b, gdst_b, _, sfi_b, sfj_b, _, si_b = b
            pltpu.make_async_copy(g.at[gdst_b], fi_b, sfi_b).wait()
            pltpu.make_async_copy(g.at[gsrc_b], fj_b, sfj_b).wait()
            pltpu.make_async_copy(idx_slice(NCH + 2 + k), idx_b, si_b).wait()
        plsc.subcore_barrier()

        if not last:
            pingpong(lambda i: acc_sh.at[rows(i)], lambda i: g.at[hrows(i)],
                     sq=True, zero_acc=True)
            plsc.subcore_barrier()
        else:
            pingpong(lambda i: acc_sh.at[rows(i)], lambda i: h2.at[hrows(i)],
                     sq=False, zero_acc=False)


def _sc_propagate(featT, epk):
    mesh = plsc.VectorSubcoreMesh(core_axis_name="c", subcore_axis_name="s")
    f32 = jnp.float32
    run = pl.kernel(
        _sc_body,
        out_type=[
            jax.ShapeDtypeStruct((2 * N_PAD, HD), f32),
            jax.ShapeDtypeStruct((2 * N_PAD, HD), f32),
            jax.ShapeDtypeStruct((N_PAD,), f32),
        ],
        mesh=mesh,
        scratch_types=[
            pltpu.VMEM_SHARED((N_PAD, HD), f32),
            pltpu.VMEM_SHARED((N_PAD,), f32),
            pltpu.VMEM((CH, HD), f32),
            pltpu.VMEM((CH, HD), f32),
            pltpu.VMEM((CH, HD), f32),
            pltpu.VMEM((CH, HD), f32),
            pltpu.VMEM((2 * CH,), jnp.int32),
            pltpu.VMEM((2 * CH,), jnp.int32),
            pltpu.VMEM((CH,), jnp.int32),
            pltpu.VMEM((CH,), jnp.int32),
            pltpu.VMEM((CH,), jnp.int32),
            pltpu.VMEM((CH,), jnp.int32),
            pltpu.VMEM((CH,), jnp.int32),
            pltpu.VMEM((CH,), jnp.int32),
            pltpu.VMEM((CH,), f32),
            pltpu.VMEM((NPT + 16,), f32),
            pltpu.VMEM((WB, HD), f32),
            pltpu.VMEM((WB, HD), f32),
            pltpu.VMEM((WB, HD), f32),
            pltpu.VMEM((NPT,), f32),
            pltpu.SemaphoreType.DMA,
            pltpu.SemaphoreType.DMA,
            pltpu.SemaphoreType.DMA,
            pltpu.SemaphoreType.DMA,
            pltpu.SemaphoreType.DMA,
            pltpu.SemaphoreType.DMA,
            pltpu.SemaphoreType.DMA,
            pltpu.SemaphoreType.DMA,
        ],
    )
    return run(featT, epk)


BN = 1000


def _tc_body(al_r, b_r, dn_r, fL_r, fR_r, g1a_r, g1b_r, h2a_r, h2b_r, W_r, o_r):
    a0 = al_r[0, 0]
    a1 = al_r[0, 1]
    a2 = al_r[0, 2]
    a1dn = a1 / dn_r[...]
    SL = a0 * fL_r[...] + a1dn * g1a_r[...] + a2 * h2a_r[...]
    SR = a0 * fR_r[...] + a1dn * g1b_r[...] + a2 * h2b_r[...]
    wl = W_r[:, :HD]
    wr = W_r[:, HD:]
    dnum = (((1,), (1,)), ((), ()))
    acc = lax.dot_general(SL, wl, dnum, precision=lax.Precision.HIGHEST,
                          preferred_element_type=jnp.float32)
    acc = acc + lax.dot_general(SR, wr, dnum, precision=lax.Precision.HIGHEST,
                                preferred_element_type=jnp.float32)
    o_r[...] = acc + (KSTEPS + 1) * b_r[...]


def _tc_combine(alpha2, b2, dn1, fL, fR, g1a, g1b, h2a, h2b, W):
    f32 = jnp.float32
    half = pl.BlockSpec((BN, HD), lambda i: (i, 0))
    fixed = lambda shape: pl.BlockSpec(shape, lambda i: (0, 0))
    return pl.pallas_call(
        _tc_body,
        grid=(N // BN,),
        in_specs=[
            fixed((1, 3)),
            fixed((1, D)),
            pl.BlockSpec((BN, 1), lambda i: (i, 0)),
            half, half, half, half, half, half,
            fixed((D, D)),
        ],
        out_specs=pl.BlockSpec((BN, D), lambda i: (i, 0)),
        out_shape=jax.ShapeDtypeStruct((N, D), f32),
    )(alpha2, b2, dn1, fL, fR, g1a, g1b, h2a, h2b, W)


def kernel(feat, edge_index, W, b, alpha):
    f32 = jnp.float32
    src = edge_index[0].astype(jnp.int32)
    dst = edge_index[1].astype(jnp.int32)
    npad = E_PAD + PADCH * CH - E
    pad = (N + (jnp.arange(npad, dtype=jnp.int32) % NPADROWS)).astype(jnp.int32)
    srcp = jnp.concatenate([src, pad])
    dstp = jnp.concatenate([dst, pad])
    epk = jnp.stack([srcp.reshape(-1, CH), dstp.reshape(-1, CH)], axis=1).reshape(-1)

    featp = jnp.pad(feat.astype(f32), ((0, N_PAD - N), (0, 0)))
    featT = jnp.transpose(featp.reshape(N_PAD, 2, HD), (1, 0, 2)).reshape(2 * N_PAD, HD)

    h2, g1, dnv = _sc_propagate(featT, epk)

    g1a, g1b = g1[:N], g1[N_PAD:N_PAD + N]
    h2a, h2b = h2[:N], h2[N_PAD:N_PAD + N]
    fL, fR = feat[:, :HD], feat[:, HD:]
    alpha2 = alpha.reshape(1, 3).astype(f32)
    b2 = b.reshape(1, D).astype(f32)
    dn1 = dnv[:N].reshape(N, 1)

    return _tc_combine(alpha2, b2, dn1, fL, fR, g1a, g1b, h2a, h2b, W.astype(f32))
